# baseline (speedup 1.0000x reference)
import functools

import jax
import jax.numpy as jnp
from jax.experimental import pallas as pl
from jax.experimental.pallas import tpu as pltpu

N_ROWS = 10000
D = 128
TILE = 5000
NJ = N_ROWS // TILE
SUB = 1000
NS = TILE // SUB
INV_N = 1.0 / N_ROWS


def _body(xs_ref, xt_ref, w_ref, hs_ref, ht_ref, loss_ref, acc_ref):
    j = pl.program_id(0)
    m = pl.program_id(1)

    @pl.when(jnp.logical_and(j == 0, m == 0))
    def _init():
        acc_ref[...] = jnp.zeros_like(acc_ref)

    def run(x_ref, w, h_out_ref, base):
        x = x_ref[pl.ds(m * SUB, SUB), :]
        h = jnp.dot(x, w, preferred_element_type=jnp.float32)
        h_out_ref[...] = h
        h2 = h * h
        h3 = h2 * h
        h4 = h2 * h2
        h5 = h4 * h
        part = jnp.concatenate(
            [
                jnp.sum(h, axis=0, keepdims=True),
                jnp.sum(h2, axis=0, keepdims=True),
                jnp.sum(h3, axis=0, keepdims=True),
                jnp.sum(h4, axis=0, keepdims=True),
                jnp.sum(h5, axis=0, keepdims=True),
            ],
            axis=0,
        )
        acc_ref[base : base + 5, :] += part

    run(xs_ref, w_ref[0], hs_ref, 0)
    run(xt_ref, w_ref[1], ht_ref, 8)

    @pl.when(jnp.logical_and(j == NJ - 1, m == NS - 1))
    def _finish():
        a = acc_ref[...] * INV_N

        def central(rows):
            m1 = rows[0:1, :]
            m2 = rows[1:2, :]
            m3 = rows[2:3, :]
            m4 = rows[3:4, :]
            m5 = rows[4:5, :]
            c2 = m2 - m1 * m1
            c3 = m3 - 3.0 * m1 * m2 + 2.0 * m1**3
            c4 = m4 - 4.0 * m1 * m3 + 6.0 * m1**2 * m2 - 3.0 * m1**4
            c5 = (
                m5
                - 5.0 * m1 * m4
                + 10.0 * m1**2 * m3
                - 10.0 * m1**3 * m2
                + 4.0 * m1**5
            )
            return m1, c2, c3, c4, c5

        s_moms = central(a[0:5, :])
        t_moms = central(a[8:13, :])
        loss = jnp.zeros((1, 1), jnp.float32)
        for s_m, t_m in zip(s_moms, t_moms):
            d = s_m - t_m
            loss = loss + jnp.sqrt(jnp.sum(d * d, keepdims=True))
        loss_ref[...] = loss


@functools.partial(jax.jit, static_argnames=())
def _run(feat_s, feat_t, w_stacked):
    kernel_fn = pl.pallas_call(
        _body,
        grid=(NJ, NS),
        in_specs=[
            pl.BlockSpec((TILE, D), lambda j, m: (j, 0)),
            pl.BlockSpec((TILE, D), lambda j, m: (j, 0)),
            pl.BlockSpec((2, D, D), lambda j, m: (0, 0, 0)),
        ],
        out_specs=[
            pl.BlockSpec((SUB, D), lambda j, m: (NS * j + m, 0)),
            pl.BlockSpec((SUB, D), lambda j, m: (NS * j + m, 0)),
            pl.BlockSpec((1, 1), lambda j, m: (0, 0)),
        ],
        out_shape=[
            jax.ShapeDtypeStruct((N_ROWS, D), jnp.float32),
            jax.ShapeDtypeStruct((N_ROWS, D), jnp.float32),
            jax.ShapeDtypeStruct((1, 1), jnp.float32),
        ],
        scratch_shapes=[pltpu.VMEM((16, D), jnp.float32)],
        compiler_params=pltpu.CompilerParams(
            dimension_semantics=("arbitrary", "arbitrary"),
        ),
    )
    return kernel_fn(feat_s, feat_t, w_stacked)


def kernel(feat_s, feat_t, W_s, W_t, edge_index):
    del edge_index
    w_stacked = jnp.stack([W_s, W_t])
    h_s, h_t, loss = _run(feat_s, feat_t, w_stacked)
    return (h_s, h_t, loss[0, 0])


# speedup vs baseline: 1.4049x; 1.4049x over previous
import functools

import jax
import jax.numpy as jnp
from jax.experimental import pallas as pl
from jax.experimental.pallas import tpu as pltpu

N_ROWS = 10000
D = 128
TILE = 5000
NJ = N_ROWS // TILE
INV_N = 1.0 / N_ROWS


def _body(xs_ref, xt_ref, w_ref, hs_ref, ht_ref, loss_ref, acc_ref):
    j = pl.program_id(0)

    @pl.when(j == 0)
    def _init():
        acc_ref[...] = jnp.zeros_like(acc_ref)

    def run(x_ref, w, h_out_ref, base):
        h = jnp.dot(x_ref[...], w, preferred_element_type=jnp.float32)
        h_out_ref[...] = h
        h2 = h * h
        h3 = h2 * h
        h4 = h2 * h2
        h5 = h4 * h
        part = jnp.concatenate(
            [
                jnp.sum(h, axis=0, keepdims=True),
                jnp.sum(h2, axis=0, keepdims=True),
                jnp.sum(h3, axis=0, keepdims=True),
                jnp.sum(h4, axis=0, keepdims=True),
                jnp.sum(h5, axis=0, keepdims=True),
            ],
            axis=0,
        )
        acc_ref[base : base + 5, :] += part

    run(xs_ref, w_ref[0], hs_ref, 0)
    run(xt_ref, w_ref[1], ht_ref, 8)

    @pl.when(j == NJ - 1)
    def _finish():
        a = acc_ref[...] * INV_N

        def central(rows):
            m1 = rows[0:1, :]
            m2 = rows[1:2, :]
            m3 = rows[2:3, :]
            m4 = rows[3:4, :]
            m5 = rows[4:5, :]
            c2 = m2 - m1 * m1
            c3 = m3 - 3.0 * m1 * m2 + 2.0 * m1**3
            c4 = m4 - 4.0 * m1 * m3 + 6.0 * m1**2 * m2 - 3.0 * m1**4
            c5 = (
                m5
                - 5.0 * m1 * m4
                + 10.0 * m1**2 * m3
                - 10.0 * m1**3 * m2
                + 4.0 * m1**5
            )
            return m1, c2, c3, c4, c5

        s_moms = central(a[0:5, :])
        t_moms = central(a[8:13, :])
        loss = jnp.zeros((1, 1), jnp.float32)
        for s_m, t_m in zip(s_moms, t_moms):
            d = s_m - t_m
            loss = loss + jnp.sqrt(jnp.sum(d * d, keepdims=True))
        loss_ref[...] = loss


@functools.partial(jax.jit, static_argnames=())
def _run(feat_s, feat_t, w_stacked):
    kernel_fn = pl.pallas_call(
        _body,
        grid=(NJ,),
        in_specs=[
            pl.BlockSpec((TILE, D), lambda j: (j, 0)),
            pl.BlockSpec((TILE, D), lambda j: (j, 0)),
            pl.BlockSpec((2, D, D), lambda j: (0, 0, 0)),
        ],
        out_specs=[
            pl.BlockSpec((TILE, D), lambda j: (j, 0)),
            pl.BlockSpec((TILE, D), lambda j: (j, 0)),
            pl.BlockSpec((1, 1), lambda j: (0, 0)),
        ],
        out_shape=[
            jax.ShapeDtypeStruct((N_ROWS, D), jnp.float32),
            jax.ShapeDtypeStruct((N_ROWS, D), jnp.float32),
            jax.ShapeDtypeStruct((1, 1), jnp.float32),
        ],
        scratch_shapes=[pltpu.VMEM((16, D), jnp.float32)],
        compiler_params=pltpu.CompilerParams(
            dimension_semantics=("arbitrary",),
        ),
    )
    return kernel_fn(feat_s, feat_t, w_stacked)


def kernel(feat_s, feat_t, W_s, W_t, edge_index):
    del edge_index
    w_stacked = jnp.stack([W_s, W_t])
    h_s, h_t, loss = _run(feat_s, feat_t, w_stacked)
    return (h_s, h_t, loss[0, 0])
